# baseline (speedup 1.0000x reference)
import functools

import jax
import jax.numpy as jnp
from jax import lax
from jax.experimental import pallas as pl
from jax.experimental.pallas import tpu as pltpu
from jax.experimental.pallas import tpu_sc as plsc

VOCAB = 1002
N_D = 64
BATCH = 4096
HIST = 200

NW = 32
BW = BATCH // NW
VPAD = 1024
NBUF = 4

_mesh = plsc.VectorSubcoreMesh(core_axis_name="c", subcore_axis_name="s")


@functools.partial(
    pl.kernel,
    mesh=_mesh,
    out_type=jax.ShapeDtypeStruct((HIST, N_D, BATCH), jnp.float32),
    scratch_types=[
        pltpu.VMEM((N_D, VPAD), jnp.float32),
        pltpu.VMEM((HIST, BW), jnp.int32),
        pltpu.VMEM((NBUF, N_D, BW), jnp.float32),
        pltpu.SemaphoreType.DMA((NBUF,)),
    ],
    compiler_params=pltpu.CompilerParams(use_tc_tiling_on_sc=True,
                                         needs_layout_passes=False),
)
def _sc_embed(idx_hbm, table_hbm, out_hbm, table_v, idx_v, blk_v, ssem):
    c = lax.axis_index("c")
    s = lax.axis_index("s")
    wid = s * 2 + c
    b0 = wid * BW
    pltpu.sync_copy(table_hbm, table_v)
    pltpu.sync_copy(idx_hbm.at[:, pl.ds(b0, BW)], idx_v)

    def fill(h, nb):
        for g in range(BW // 16):
            idx16 = idx_v[h, pl.ds(g * 16, 16)]

            @plsc.parallel_loop(0, N_D, unroll=16)
            def _(d):
                d16 = jnp.full((16,), d, jnp.int32)
                v = plsc.load_gather(table_v, [d16, idx16])
                blk_v[nb, d, pl.ds(g * 16, 16)] = v

    def store(h, nb):
        pltpu.async_copy(blk_v.at[nb], out_hbm.at[h, :, pl.ds(b0, BW)],
                         ssem.at[nb])

    def wait_store(h, nb):
        pltpu.make_async_copy(blk_v.at[nb], out_hbm.at[h, :, pl.ds(b0, BW)],
                              ssem.at[nb]).wait()

    for nb in range(NBUF):
        fill(nb, nb)
        store(nb, nb)

    def group(g, carry):
        base = g * NBUF
        for nb in range(NBUF):
            h = base + nb
            wait_store(h, nb)
            fill(h + NBUF, nb)
            store(h + NBUF, nb)
        return carry

    lax.fori_loop(0, HIST // NBUF - 1, group, 0, unroll=False)

    last = HIST - NBUF
    for nb in range(NBUF):
        wait_store(last + nb, nb)


def kernel(input, table):
    idx_t = input.T.astype(jnp.int32)
    tbl_t = jnp.pad(table.T, ((0, 0), (0, VPAD - VOCAB)))
    out_phys = _sc_embed(idx_t, tbl_t)
    return jnp.transpose(out_phys, (2, 0, 1))


# speedup vs baseline: 1.3232x; 1.3232x over previous
import functools

import jax
import jax.numpy as jnp
from jax import lax
from jax.experimental import pallas as pl
from jax.experimental.pallas import tpu as pltpu
from jax.experimental.pallas import tpu_sc as plsc

VOCAB = 1002
N_D = 64
BATCH = 4096
HIST = 200

NW = 32
BW = BATCH // NW
VPAD = 1024
NBUF = 2

_mesh = plsc.VectorSubcoreMesh(core_axis_name="c", subcore_axis_name="s")


@functools.partial(
    pl.kernel,
    mesh=_mesh,
    out_type=jax.ShapeDtypeStruct((HIST, N_D, BATCH), jnp.float32),
    scratch_types=[
        pltpu.VMEM((N_D * VPAD,), jnp.float32),
        pltpu.VMEM((HIST, BW), jnp.int32),
        pltpu.VMEM((NBUF, N_D, BW), jnp.float32),
        pltpu.SemaphoreType.DMA((NBUF,)),
    ],
    compiler_params=pltpu.CompilerParams(use_tc_tiling_on_sc=True,
                                         needs_layout_passes=False),
)
def _sc_embed(idx_hbm, table_hbm, out_hbm, table_v, idx_v, blk_v, ssem):
    c = lax.axis_index("c")
    s = lax.axis_index("s")
    wid = s * 2 + c
    b0 = wid * BW
    pltpu.sync_copy(table_hbm, table_v)
    pltpu.sync_copy(idx_hbm.at[:, pl.ds(b0, BW)], idx_v)

    def fill(h, nb):
        for g in range(BW // 16):
            idx16 = idx_v[h, pl.ds(g * 16, 16)]

            @plsc.parallel_loop(0, N_D, unroll=16)
            def _(d):
                v = plsc.load_gather(table_v, [idx16 + (d << 10)])
                blk_v[nb, d, pl.ds(g * 16, 16)] = v

    def store(h, nb):
        pltpu.async_copy(blk_v.at[nb], out_hbm.at[h, :, pl.ds(b0, BW)],
                         ssem.at[nb])

    def wait_store(h, nb):
        pltpu.make_async_copy(blk_v.at[nb], out_hbm.at[h, :, pl.ds(b0, BW)],
                              ssem.at[nb]).wait()

    for nb in range(NBUF):
        fill(nb, nb)
        store(nb, nb)

    def group(g, carry):
        base = g * NBUF
        for nb in range(NBUF):
            h = base + nb
            wait_store(h, nb)
            fill(h + NBUF, nb)
            store(h + NBUF, nb)
        return carry

    lax.fori_loop(0, HIST // NBUF - 1, group, 0, unroll=False)

    last = HIST - NBUF
    for nb in range(NBUF):
        wait_store(last + nb, nb)


def kernel(input, table):
    idx_t = input.T.astype(jnp.int32)
    tbl_t = jnp.pad(table.T, ((0, 0), (0, VPAD - VOCAB)))
    out_phys = _sc_embed(idx_t, tbl_t.reshape(-1))
    return jnp.transpose(out_phys, (2, 0, 1))


# speedup vs baseline: 1.3305x; 1.0055x over previous
import functools

import jax
import jax.numpy as jnp
from jax import lax
from jax.experimental import pallas as pl
from jax.experimental.pallas import tpu as pltpu
from jax.experimental.pallas import tpu_sc as plsc

VOCAB = 1002
N_D = 64
BATCH = 4096
HIST = 200

NW = 32
BW = BATCH // NW
VPAD = 1024
NBUF = 2

_mesh = plsc.VectorSubcoreMesh(core_axis_name="c", subcore_axis_name="s")


@functools.partial(
    pl.kernel,
    mesh=_mesh,
    out_type=jax.ShapeDtypeStruct((HIST, N_D, BATCH), jnp.float32),
    scratch_types=[
        pltpu.VMEM((N_D * VPAD,), jnp.float32),
        pltpu.VMEM((HIST, BW), jnp.int32),
        pltpu.VMEM((NBUF, N_D, BW), jnp.float32),
        pltpu.SemaphoreType.DMA((NBUF,)),
        pltpu.SemaphoreType.DMA((2,)),
    ],
    compiler_params=pltpu.CompilerParams(use_tc_tiling_on_sc=True,
                                         needs_layout_passes=False),
)
def _sc_embed(idx_hbm, table_hbm, out_hbm, table_v, idx_v, blk_v, ssem,
              lsem):
    c = lax.axis_index("c")
    s = lax.axis_index("s")
    wid = s * 2 + c
    b0 = wid * BW
    tcopy = pltpu.async_copy(table_hbm, table_v, lsem.at[0])
    icopy = pltpu.async_copy(idx_hbm.at[:, pl.ds(b0, BW)], idx_v, lsem.at[1])
    tcopy.wait()
    icopy.wait()

    def fill(h, nb):
        for g in range(BW // 16):
            idx16 = idx_v[h, pl.ds(g * 16, 16)]

            @plsc.parallel_loop(0, N_D, unroll=16)
            def _(d):
                v = plsc.load_gather(table_v, [idx16 + (d << 10)])
                blk_v[nb, d, pl.ds(g * 16, 16)] = v

    def store(h, nb):
        pltpu.async_copy(blk_v.at[nb], out_hbm.at[h, :, pl.ds(b0, BW)],
                         ssem.at[nb])

    def wait_store(h, nb):
        pltpu.make_async_copy(blk_v.at[nb], out_hbm.at[h, :, pl.ds(b0, BW)],
                              ssem.at[nb]).wait()

    for nb in range(NBUF):
        fill(nb, nb)
        store(nb, nb)

    def group(g, carry):
        base = g * NBUF
        for nb in range(NBUF):
            h = base + nb
            wait_store(h, nb)
            fill(h + NBUF, nb)
            store(h + NBUF, nb)
        return carry

    lax.fori_loop(0, HIST // NBUF - 1, group, 0, unroll=False)

    last = HIST - NBUF
    for nb in range(NBUF):
        wait_store(last + nb, nb)


def kernel(input, table):
    idx_t = input.T.astype(jnp.int32)
    tbl_t = jnp.pad(table.T, ((0, 0), (0, VPAD - VOCAB)))
    out_phys = _sc_embed(idx_t, tbl_t.reshape(-1))
    return jnp.transpose(out_phys, (2, 0, 1))
